# baseline (speedup 1.0000x reference)
import functools

import jax
import jax.numpy as jnp
from jax.experimental import pallas as pl
from jax.experimental.pallas import tpu as pltpu

_R = 32
_NQ = 4
_CQ = 25088


def _margin_kernel(k_ref, p0, p1, p2, p3, out_ref, pk_acc, *, C):
    i = pl.program_id(0)
    refs = (p0, p1, p2, p3)
    lane = jax.lax.broadcasted_iota(jnp.int32, (1, 128), 1)

    for r in range(_R):
        c = k_ref[i * _R + r]
        for q in range(_NQ):
            def _rmw(q=q, ref=refs[q], c=c, r=r):
                c_loc = c - q * _CQ
                c0 = (c_loc // 128) * 128
                chunk = ref[pl.ds(r, 1), pl.ds(c0, 128)]
                is_l = lane == (c_loc - c0)
                pk_acc[pl.ds(r, 1), :] = jnp.where(is_l, chunk, -jnp.inf).max(
                    axis=1, keepdims=True)
                ref[pl.ds(r, 1), pl.ds(c0, 128)] = jnp.where(is_l, -jnp.inf, chunk)
            pl.when((c >= q * _CQ) & (c < (q + 1) * _CQ))(_rmw)

    m = jnp.maximum(jnp.maximum(p0[...].max(axis=1), p1[...].max(axis=1)),
                    p2[...].max(axis=1))
    last_lo = (_NQ - 1) * _CQ
    nvalid = C - last_lo
    n_al = (nvalid // 128) * 128
    m = jnp.maximum(m, p3[:, :n_al].max(axis=1))
    tail = p3[:, n_al:n_al + 128]
    tmask = jax.lax.broadcasted_iota(jnp.int32, tail.shape, 1) < (nvalid - n_al)
    m = jnp.maximum(m, jnp.where(tmask, tail, -jnp.inf).max(axis=1))

    out_ref[...] = m[:, None] - pk_acc[...]


def kernel(prediction, k):
    B, C = prediction.shape
    k2 = k.astype(jnp.int32)
    pred_specs = [
        pl.BlockSpec((_R, _CQ), lambda i, q=q: (i, q)) for q in range(_NQ)
    ]
    out = pl.pallas_call(
        functools.partial(_margin_kernel, C=C),
        grid=(B // _R,),
        in_specs=[pl.BlockSpec(memory_space=pltpu.SMEM)] + pred_specs,
        out_specs=pl.BlockSpec((_R, 1), lambda i: (i, 0)),
        out_shape=jax.ShapeDtypeStruct((B, 1), jnp.float32),
        scratch_shapes=[pltpu.VMEM((_R, 1), jnp.float32)],
        compiler_params=pltpu.CompilerParams(
            dimension_semantics=("arbitrary",),
        ),
    )(k2, prediction, prediction, prediction, prediction)
    return out.reshape(B)
